# baseline (speedup 1.0000x reference)
import functools

import jax
import jax.numpy as jnp
from jax import lax
from jax.experimental import pallas as pl
from jax.experimental.pallas import tpu as pltpu
from jax.experimental.pallas import tpu_sc as plsc

B = 16384
D = 16
_NC = 2
_NS = 16
_NW = _NC * _NS
_BPW = B // _NW

_sc_mesh = plsc.VectorSubcoreMesh(core_axis_name="c", subcore_axis_name="s")


@functools.partial(
    pl.kernel,
    mesh=_sc_mesh,
    out_type=[
        jax.ShapeDtypeStruct((B, 128), jnp.float32),
        jax.ShapeDtypeStruct((B, 128), jnp.float32),
    ],
    scratch_types=[
        pltpu.VMEM((_BPW,), jnp.int32),
        pltpu.VMEM((_BPW, 128), jnp.float32),
        pltpu.SemaphoreType.DMA,
    ],
)
def _sc_gather(wr_hbm, g0_hbm, g1_hbm, u8_hbm, v8_hbm, g_v, rows_v, sem):
    wid = lax.axis_index("s") * _NC + lax.axis_index("c")
    base = wid * _BPW
    pltpu.sync_copy(g0_hbm.at[pl.ds(base, _BPW)], g_v)
    pltpu.async_copy(wr_hbm.at[g_v], rows_v, sem).wait()
    pltpu.sync_copy(rows_v, u8_hbm.at[pl.ds(base, _BPW)])
    pltpu.sync_copy(g1_hbm.at[pl.ds(base, _BPW)], g_v)
    pltpu.async_copy(wr_hbm.at[g_v], rows_v, sem).wait()
    pltpu.sync_copy(rows_v, v8_hbm.at[pl.ds(base, _BPW)])


_R = 2048
_G = B // _R


def _tc_body(u8_ref, v8_ref, a0_ref, a1_ref, val_ref, out_ref, *, pairs):
    g = pl.program_id(0)

    @pl.when(g == 0)
    def _init():
        out_ref[0, 0] = 0.0

    u8 = u8_ref[...]
    v8 = v8_ref[...]
    a0 = a0_ref[...]
    a1 = a1_ref[...]
    lane = lax.broadcasted_iota(jnp.int32, (_R, 128), 1)
    m0 = (lane >> 4) == a0
    m1 = (lane >> 4) == a1
    perm = (lane + 16 * (a1 - a0)) & 127
    va = jnp.take_along_axis(v8, perm, axis=1)
    du = jnp.where(m0, u8 - va, 0.0)
    um = jnp.where(m0, u8, 0.0)
    vm = jnp.where(m1, v8, 0.0)
    z = 2.0 * jnp.sum(du * du, axis=1, keepdims=True)
    su = jnp.sum(um * um, axis=1, keepdims=True)
    sv = jnp.sum(vm * vm, axis=1, keepdims=True)
    uu = 1.0 + z / ((1.0 - su) * (1.0 - sv))
    dist = jnp.log(uu + jnp.sqrt(uu * uu - 1.0))
    r = dist - val_ref[...]
    out_ref[0, 0] += jnp.sum(r * r) / pairs


def kernel(idx, values, w):
    n = w.shape[0]
    pairs = n * (n - 1) / 2.0
    idx32 = idx.astype(jnp.int32)
    i0 = idx32[:, 0]
    i1 = idx32[:, 1]
    wr = w.reshape(n // 8, 8 * D)
    u8, v8 = _sc_gather(wr, i0 >> 3, i1 >> 3)
    loss = pl.pallas_call(
        functools.partial(_tc_body, pairs=pairs),
        grid=(_G,),
        out_shape=jax.ShapeDtypeStruct((1, 1), jnp.float32),
        in_specs=[
            pl.BlockSpec((_R, 128), lambda g: (g, 0)),
            pl.BlockSpec((_R, 128), lambda g: (g, 0)),
            pl.BlockSpec((_R, 1), lambda g: (g, 0)),
            pl.BlockSpec((_R, 1), lambda g: (g, 0)),
            pl.BlockSpec((_R, 1), lambda g: (g, 0)),
        ],
        out_specs=pl.BlockSpec((1, 1), lambda g: (0, 0),
                               memory_space=pltpu.SMEM),
    )(u8, v8, (i0 & 7).reshape(B, 1), (i1 & 7).reshape(B, 1),
      values.reshape(B, 1))
    return loss[0, 0]


# speedup vs baseline: 1.0853x; 1.0853x over previous
import functools

import jax
import jax.numpy as jnp
from jax import lax
from jax.experimental import pallas as pl
from jax.experimental.pallas import tpu as pltpu
from jax.experimental.pallas import tpu_sc as plsc

B = 16384
D = 16
_NC = 2
_NS = 16
_NW = _NC * _NS
_BPW = B // _NW

_sc_mesh = plsc.VectorSubcoreMesh(core_axis_name="c", subcore_axis_name="s")


@functools.partial(
    pl.kernel,
    mesh=_sc_mesh,
    compiler_params=pltpu.CompilerParams(use_tc_tiling_on_sc=False),
    out_type=[
        jax.ShapeDtypeStruct((B, D), jnp.float32),
        jax.ShapeDtypeStruct((B, D), jnp.float32),
    ],
    scratch_types=[
        pltpu.VMEM((_BPW,), jnp.int32),
        pltpu.VMEM((_BPW,), jnp.int32),
        pltpu.VMEM((_BPW, D), jnp.float32),
        pltpu.VMEM((_BPW, D), jnp.float32),
        pltpu.SemaphoreType.DMA,
    ],
)
def _sc_gather(w_hbm, g0_hbm, g1_hbm, u_hbm, v_hbm, g0_v, g1_v, r0_v, r1_v,
               sem):
    wid = lax.axis_index("s") * _NC + lax.axis_index("c")
    base = wid * _BPW
    pltpu.sync_copy(g0_hbm.at[pl.ds(base, _BPW)], g0_v)
    pltpu.sync_copy(g1_hbm.at[pl.ds(base, _BPW)], g1_v)
    c0 = pltpu.async_copy(w_hbm.at[g0_v], r0_v, sem)
    c1 = pltpu.async_copy(w_hbm.at[g1_v], r1_v, sem)
    c0.wait()
    c1.wait()
    pltpu.sync_copy(r0_v, u_hbm.at[pl.ds(base, _BPW)])
    pltpu.sync_copy(r1_v, v_hbm.at[pl.ds(base, _BPW)])


_R = B // 8


def _tc_body(u_ref, v_ref, val_ref, out_ref, *, pairs):
    u = u_ref[...]
    v = v_ref[...]
    du = u - v
    lane = lax.broadcasted_iota(jnp.int32, (128, 8), 0)
    grp = lax.broadcasted_iota(jnp.int32, (128, 8), 1)
    m = ((lane >> 4) == grp).astype(jnp.float32)
    z = lax.dot(du * du, m, preferred_element_type=jnp.float32)
    su = lax.dot(u * u, m, preferred_element_type=jnp.float32)
    sv = lax.dot(v * v, m, preferred_element_type=jnp.float32)
    uu = 1.0 + 2.0 * z / ((1.0 - su) * (1.0 - sv))
    dist = jnp.log(uu + jnp.sqrt(uu * uu - 1.0))
    r = dist - val_ref[...]
    out_ref[0, 0] = jnp.sum(r * r) / pairs


def kernel(idx, values, w):
    n = w.shape[0]
    pairs = n * (n - 1) / 2.0
    idx32 = idx.astype(jnp.int32)
    u, v = _sc_gather(w, idx32[:, 0], idx32[:, 1])
    loss = pl.pallas_call(
        functools.partial(_tc_body, pairs=pairs),
        out_shape=jax.ShapeDtypeStruct((1, 1), jnp.float32),
        in_specs=[
            pl.BlockSpec((_R, 128), lambda: (0, 0)),
            pl.BlockSpec((_R, 128), lambda: (0, 0)),
            pl.BlockSpec((_R, 8), lambda: (0, 0)),
        ],
        out_specs=pl.BlockSpec((1, 1), lambda: (0, 0),
                               memory_space=pltpu.SMEM),
    )(u.reshape(_R, 128), v.reshape(_R, 128), values.reshape(_R, 8))
    return loss[0, 0]
